# baseline (speedup 1.0000x reference)
import functools

import jax
import jax.numpy as jnp
from jax import lax
from jax.experimental import pallas as pl
from jax.experimental.pallas import tpu as pltpu
from jax.experimental.pallas import tpu_sc as plsc

_CHUNK = 128
_LANES = 16
_PACK = 8


def _make_gather(B, D):
    info = plsc.get_sparse_core_info()
    NC, NS = info.num_cores, info.num_subcores
    NW = NC * NS
    b_per_w = B // NW
    n_chunks = b_per_w // _CHUNK

    mesh = plsc.VectorSubcoreMesh(core_axis_name="c", subcore_axis_name="s")

    @functools.partial(
        pl.kernel,
        mesh=mesh,
        out_type=jax.ShapeDtypeStruct((B, D), jnp.float32),
        scratch_types=[
            pltpu.VMEM((n_chunks, _CHUNK), jnp.int32),
            pltpu.VMEM((n_chunks, _CHUNK), jnp.int32),
            pltpu.VMEM((n_chunks, _CHUNK), jnp.int32),
            pltpu.VMEM((2, _CHUNK, _PACK * D), jnp.float32),
            pltpu.VMEM((2, _CHUNK, D), jnp.float32),
            pltpu.SemaphoreType.DMA,
            pltpu.SemaphoreType.DMA,
        ],
        compiler_params=pltpu.CompilerParams(needs_layout_passes=False),
    )
    def gather_kernel(idx_hbm, lines_hbm, out_hbm,
                      idx_v, row_v, off_v, lines_v, out_v, sem0, sem1):
        sems = (sem0, sem1)
        wid = lax.axis_index("s") * NC + lax.axis_index("c")
        base = wid * b_per_w
        pltpu.sync_copy(idx_hbm.at[wid], idx_v)

        for c in range(n_chunks):
            for k in range(_CHUNK // _LANES):
                sl = pl.ds(k * _LANES, _LANES)
                v = idx_v.at[c][sl]
                row_v.at[c][sl] = lax.shift_right_logical(v, 3)
                off_v.at[c][sl] = lax.shift_left(
                    jnp.bitwise_and(v, _PACK - 1), 4)

        def fire(c):
            return pltpu.async_copy(
                lines_hbm.at[row_v.at[c]], lines_v.at[c % 2], sems[c % 2])

        lane_ids = lax.iota(jnp.int32, _LANES)
        copies = {0: fire(0), 1: fire(1)}
        for c in range(n_chunks):
            copies[c].wait()
            buf = lines_v.at[c % 2]
            obuf = out_v.at[c % 2]
            for g in range(_CHUNK // _LANES):
                j_vec = g * _LANES + lane_ids
                offs = off_v.at[c][pl.ds(g * _LANES, _LANES)]
                for l in range(D):
                    vals = plsc.load_gather(buf, [j_vec, offs + l])
                    plsc.store_scatter(
                        obuf, [j_vec, jnp.full((_LANES,), l, jnp.int32)],
                        vals)
            if c + 2 < n_chunks:
                copies[c + 2] = fire(c + 2)
            pltpu.sync_copy(obuf, out_hbm.at[pl.ds(base + c * _CHUNK, _CHUNK)])

    return gather_kernel


def kernel(user_inputs, table):
    B, = user_inputs.shape
    V, D = table.shape
    info = plsc.get_sparse_core_info()
    NW = info.num_cores * info.num_subcores
    b_per_w = B // NW
    lines = table.reshape(V // _PACK, _PACK * D)
    idx3 = user_inputs.astype(jnp.int32).reshape(NW, b_per_w // _CHUNK, _CHUNK)
    return _make_gather(B, D)(idx3, lines)


# speedup vs baseline: 1.0169x; 1.0169x over previous
import functools

import jax
import jax.numpy as jnp
from jax import lax
from jax.experimental import pallas as pl
from jax.experimental.pallas import tpu as pltpu
from jax.experimental.pallas import tpu_sc as plsc


def _make_gather(B, V, D):
    info = plsc.get_sparse_core_info()
    NC, NS = info.num_cores, info.num_subcores
    NW = NC * NS
    b_per_w = B // NW

    mesh = plsc.VectorSubcoreMesh(core_axis_name="c", subcore_axis_name="s")

    @functools.partial(
        pl.kernel,
        mesh=mesh,
        out_type=jax.ShapeDtypeStruct((B, D), jnp.float32),
        scratch_types=[
            pltpu.VMEM((b_per_w,), jnp.int32),
            pltpu.VMEM((b_per_w, D), jnp.float32),
            pltpu.SemaphoreType.DMA,
        ],
        compiler_params=pltpu.CompilerParams(use_tc_tiling_on_sc=False),
    )
    def gather_kernel(table_hbm, idx_hbm, out_hbm, idx_v, rows_v, sem):
        wid = lax.axis_index("s") * NC + lax.axis_index("c")
        base = wid * b_per_w
        pltpu.sync_copy(idx_hbm.at[pl.ds(base, b_per_w)], idx_v)
        pltpu.async_copy(table_hbm.at[idx_v], rows_v, sem).wait()
        pltpu.sync_copy(rows_v, out_hbm.at[pl.ds(base, b_per_w)])

    return gather_kernel


def kernel(user_inputs, table):
    B, = user_inputs.shape
    V, D = table.shape
    idx = user_inputs.astype(jnp.int32)
    return _make_gather(B, V, D)(table, idx)


# speedup vs baseline: 1.6872x; 1.6591x over previous
import functools

import jax
import jax.numpy as jnp
from jax import lax
from jax.experimental import pallas as pl
from jax.experimental.pallas import tpu as pltpu
from jax.experimental.pallas import tpu_sc as plsc


def _make_gather(B, D):
    info = plsc.get_sparse_core_info()
    NC, NS = info.num_cores, info.num_subcores
    NW = NC * NS
    b_per_w = B // NW

    mesh = plsc.VectorSubcoreMesh(core_axis_name="c", subcore_axis_name="s")

    @functools.partial(
        pl.kernel,
        mesh=mesh,
        out_type=jax.ShapeDtypeStruct((B, D), jnp.float32),
        scratch_types=[
            pltpu.VMEM((b_per_w,), jnp.int32),
            pltpu.VMEM((b_per_w, D), jnp.float32),
            pltpu.SemaphoreType.DMA,
        ],
        compiler_params=pltpu.CompilerParams(needs_layout_passes=False),
    )
    def gather_kernel(idx_hbm, table_hbm, out_hbm, idx_v, rows_v, sem):
        wid = lax.axis_index("s") * NC + lax.axis_index("c")
        base = wid * b_per_w
        pltpu.sync_copy(idx_hbm.at[pl.ds(base, b_per_w)], idx_v)

        def body(g):
            v = idx_v[pl.ds(g * 16, 16)]
            for l in range(16):
                pltpu.async_copy(
                    table_hbm.at[pl.ds(v[l], 1)],
                    rows_v.at[pl.ds(g * 16 + l, 1)], sem)

        pl.loop(0, b_per_w // 16)(body)
        pltpu.make_async_copy(
            table_hbm.at[pl.ds(0, b_per_w)], rows_v, sem).wait()
        pltpu.sync_copy(rows_v, out_hbm.at[pl.ds(base, b_per_w)])

    return gather_kernel


def kernel(user_inputs, table):
    B, = user_inputs.shape
    V, D = table.shape
    idx = user_inputs.astype(jnp.int32)
    return _make_gather(B, D)(idx, table)
